# Initial kernel scaffold; baseline (speedup 1.0000x reference)
import functools

import jax
import jax.numpy as jnp
from jax import lax
from jax.experimental import pallas as pl
from jax.experimental.pallas import tpu as pltpu
from jax.experimental.pallas import tpu_sc as plsc


def _make_gather(NB, V, D, NC, NS):
    NW = NC * NS
    BPW = NB // NW
    C = 64
    NCHUNK = BPW // C

    mesh = plsc.VectorSubcoreMesh(core_axis_name="c", subcore_axis_name="s")

    @functools.partial(
        pl.kernel,
        mesh=mesh,
        out_type=jax.ShapeDtypeStruct((NB, D), jnp.float32),
        scratch_types=[
            pltpu.VMEM((BPW,), jnp.int32),
            pltpu.VMEM((C, D), jnp.float32),
            pltpu.SemaphoreType.DMA,
        ],
    )
    def gather_kernel(idx_hbm, table_hbm, out_hbm, idx_v, rows_v, sem):
        wid = lax.axis_index("s") * NC + lax.axis_index("c")
        base = wid * BPW
        pltpu.sync_copy(idx_hbm.at[pl.ds(base, BPW)], idx_v)

        def body(i, _):
            pltpu.async_copy(
                table_hbm.at[idx_v.at[pl.ds(i * C, C)]], rows_v, sem
            ).wait()
            pltpu.sync_copy(rows_v, out_hbm.at[pl.ds(base + i * C, C)])
            return 0

        lax.fori_loop(0, NCHUNK, body, 0)

    return gather_kernel


def kernel(indices, table):
    B, H = indices.shape
    V, D = table.shape
    NB = B * H
    flat_idx = indices.reshape(NB).astype(jnp.int32)
    info = plsc.get_sparse_core_info()
    out = _make_gather(NB, V, D, info.num_cores, info.num_subcores)(
        flat_idx, table
    )
    return out.reshape(B, H, D)


# speedup vs baseline: 1.4064x; 1.4064x over previous
import functools

import jax
import jax.numpy as jnp
from jax import lax
from jax.experimental import pallas as pl
from jax.experimental.pallas import tpu as pltpu
from jax.experimental.pallas import tpu_sc as plsc


def _make_gather(NB, V, D, NC, NS):
    NW = NC * NS
    BPW = NB // NW
    C = 64
    NCHUNK = BPW // C

    mesh = plsc.VectorSubcoreMesh(core_axis_name="c", subcore_axis_name="s")

    @functools.partial(
        pl.kernel,
        mesh=mesh,
        out_type=jax.ShapeDtypeStruct((NB, D), jnp.float32),
        scratch_types=[
            pltpu.VMEM((BPW,), jnp.int32),
            pltpu.VMEM((C, D), jnp.float32),
            pltpu.SemaphoreType.DMA,
        ],
        compiler_params=pltpu.CompilerParams(use_tc_tiling_on_sc=False),
    )
    def gather_kernel(idx_hbm, table_hbm, out_hbm, idx_v, rows_v, sem):
        wid = lax.axis_index("s") * NC + lax.axis_index("c")
        base = wid * BPW
        pltpu.sync_copy(idx_hbm.at[pl.ds(base, BPW)], idx_v)

        def body(i, _):
            pltpu.async_copy(
                table_hbm.at[idx_v.at[pl.ds(i * C, C)]], rows_v, sem
            ).wait()
            pltpu.sync_copy(rows_v, out_hbm.at[pl.ds(base + i * C, C)])
            return 0

        lax.fori_loop(0, NCHUNK, body, 0)

    return gather_kernel


def kernel(indices, table):
    B, H = indices.shape
    V, D = table.shape
    NB = B * H
    flat_idx = indices.reshape(NB).astype(jnp.int32)
    info = plsc.get_sparse_core_info()
    out = _make_gather(NB, V, D, info.num_cores, info.num_subcores)(
        flat_idx, table
    )
    return out.reshape(B, H, D)


# speedup vs baseline: 1.4256x; 1.0137x over previous
import functools

import jax
import jax.numpy as jnp
from jax import lax
from jax.experimental import pallas as pl
from jax.experimental.pallas import tpu as pltpu
from jax.experimental.pallas import tpu_sc as plsc


def _make_gather(NB, V, D, NC, NS):
    NW = NC * NS
    BPW = NB // NW
    C = 32
    NBUF = 4
    NCHUNK = BPW // C
    NROUND = NCHUNK // NBUF

    mesh = plsc.VectorSubcoreMesh(core_axis_name="c", subcore_axis_name="s")

    @functools.partial(
        pl.kernel,
        mesh=mesh,
        out_type=jax.ShapeDtypeStruct((NB, D), jnp.float32),
        scratch_types=[
            pltpu.VMEM((BPW,), jnp.int32),
            [pltpu.VMEM((C, D), jnp.float32)] * NBUF,
            [pltpu.SemaphoreType.DMA] * NBUF,
            [pltpu.SemaphoreType.DMA] * NBUF,
        ],
        compiler_params=pltpu.CompilerParams(use_tc_tiling_on_sc=False),
    )
    def gather_kernel(idx_hbm, table_hbm, out_hbm, idx_v, rows, gsems, osems):
        wid = lax.axis_index("s") * NC + lax.axis_index("c")
        base = wid * BPW
        pltpu.sync_copy(idx_hbm.at[pl.ds(base, BPW)], idx_v)

        def gather_desc(i, b):
            return pltpu.make_async_copy(
                table_hbm.at[idx_v.at[pl.ds(i * C, C)]], rows[b], gsems[b]
            )

        def out_desc(i, b):
            return pltpu.make_async_copy(
                rows[b], out_hbm.at[pl.ds(base + i * C, C)], osems[b]
            )

        for b in range(NBUF):
            gather_desc(b, b).start()

        def body(p, _):
            i0 = p * NBUF
            for b in range(NBUF):
                gather_desc(i0 + b, b).wait()
                out_desc(i0 + b, b).start()
            for b in range(NBUF):
                out_desc(i0 + b, b).wait()
                gather_desc(i0 + NBUF + b, b).start()
            return 0

        lax.fori_loop(0, NROUND - 1, body, 0)

        i0 = (NROUND - 1) * NBUF
        for b in range(NBUF):
            gather_desc(i0 + b, b).wait()
            out_desc(i0 + b, b).start()
        for b in range(NBUF):
            out_desc(i0 + b, b).wait()

    return gather_kernel


def kernel(indices, table):
    B, H = indices.shape
    V, D = table.shape
    NB = B * H
    flat_idx = indices.reshape(NB).astype(jnp.int32)
    info = plsc.get_sparse_core_info()
    out = _make_gather(NB, V, D, info.num_cores, info.num_subcores)(
        flat_idx, table
    )
    return out.reshape(B, H, D)


# speedup vs baseline: 1.5082x; 1.0580x over previous
import functools

import jax
import jax.numpy as jnp
from jax import lax
from jax.experimental import pallas as pl
from jax.experimental.pallas import tpu as pltpu
from jax.experimental.pallas import tpu_sc as plsc


def _make_gather(NB, V, D, NC, NS):
    NW = NC * NS
    BPW = NB // NW
    C = 32
    NBUF = 2
    NCHUNK = BPW // C
    NROUND = NCHUNK // NBUF

    mesh = plsc.VectorSubcoreMesh(core_axis_name="c", subcore_axis_name="s")

    @functools.partial(
        pl.kernel,
        mesh=mesh,
        out_type=jax.ShapeDtypeStruct((NB, D), jnp.float32),
        scratch_types=[
            pltpu.VMEM((BPW,), jnp.int32),
            [pltpu.VMEM((C, D), jnp.float32)] * NBUF,
            [pltpu.SemaphoreType.DMA] * NBUF,
            [pltpu.SemaphoreType.DMA] * NBUF,
            pltpu.VMEM_SHARED((V, D), jnp.float32),
        ],
        compiler_params=pltpu.CompilerParams(use_tc_tiling_on_sc=False),
    )
    def gather_kernel(
        idx_hbm, table_hbm, out_hbm, idx_v, rows, gsems, osems, table_sh
    ):
        wid = lax.axis_index("s") * NC + lax.axis_index("c")
        base = wid * BPW

        @pl.when(lax.axis_index("s") == 0)
        def _stage():
            pltpu.sync_copy(table_hbm, table_sh)

        pltpu.sync_copy(idx_hbm.at[pl.ds(base, BPW)], idx_v)
        plsc.subcore_barrier()

        def gather_desc(i, b):
            return pltpu.make_async_copy(
                table_sh.at[idx_v.at[pl.ds(i * C, C)]], rows[b], gsems[b]
            )

        def out_desc(i, b):
            return pltpu.make_async_copy(
                rows[b], out_hbm.at[pl.ds(base + i * C, C)], osems[b]
            )

        for b in range(NBUF):
            gather_desc(b, b).start()

        def body(p, _):
            i0 = p * NBUF
            for b in range(NBUF):
                gather_desc(i0 + b, b).wait()
                out_desc(i0 + b, b).start()
            for b in range(NBUF):
                out_desc(i0 + b, b).wait()
                gather_desc(i0 + NBUF + b, b).start()
            return 0

        lax.fori_loop(0, NROUND - 1, body, 0)

        i0 = (NROUND - 1) * NBUF
        for b in range(NBUF):
            gather_desc(i0 + b, b).wait()
            out_desc(i0 + b, b).start()
        for b in range(NBUF):
            out_desc(i0 + b, b).wait()

    return gather_kernel


def kernel(indices, table):
    B, H = indices.shape
    V, D = table.shape
    NB = B * H
    flat_idx = indices.reshape(NB).astype(jnp.int32)
    info = plsc.get_sparse_core_info()
    out = _make_gather(NB, V, D, info.num_cores, info.num_subcores)(
        flat_idx, table
    )
    return out.reshape(B, H, D)
